# baseline (speedup 1.0000x reference)
import math

import jax
import jax.numpy as jnp
from jax.experimental import pallas as pl

N = 10000
D = 256
H = 256
G = 64
OUT = 2
R = 8
B = N // R
NEG_INF = float("-inf")


def _dot_rt(a, b):
    return jax.lax.dot_general(a, b, (((1,), (1,)), ((), ())),
                               preferred_element_type=jnp.float32)


def _seg_scan(vals, ids, nrows, nsteps):
    f32 = jnp.float32
    w = vals.shape[1]
    for k in range(nsteps):
        s = 1 << k
        same = (jax.lax.slice(ids, (0, 0), (nrows - s, 1)) ==
                jax.lax.slice(ids, (s, 0), (nrows, 1)))
        prev = jax.lax.slice(vals, (0, 0), (nrows - s, w))
        cur = jax.lax.slice(vals, (s, 0), (nrows, w))
        rest = jnp.maximum(cur, jnp.where(same, prev, NEG_INF))
        vals = jnp.concatenate(
            [jax.lax.slice(vals, (0, 0), (s, w)), rest], axis=0)
    return vals


def _fused_kernel(x_ref, batch_col_ref, batch_row_ref,
                  w1_ref, wc1_ref, pk_ref,
                  out_ref):
    f32 = jnp.float32

    pk = pk_ref[...]
    h = _dot_rt(x_ref[...], w1_ref[...]) + jax.lax.slice(pk, (136, 0), (137, H))

    batch_col = batch_col_ref[...]
    batch_row = batch_row_ref[...]

    seg_iota = jax.lax.broadcasted_iota(jnp.int32, (G, 1), 0)
    eq = (batch_row == seg_iota).astype(f32)

    counts = jnp.sum(eq, axis=1, keepdims=True)
    tri = (seg_iota >= jax.lax.broadcasted_iota(jnp.int32, (1, G), 1))
    tri = tri.astype(f32)
    ends = jnp.dot(tri, counts,
                   preferred_element_type=f32).astype(jnp.int32) - 1

    x_sum = jnp.dot(eq, h, preferred_element_type=f32)

    bf16 = jnp.bfloat16
    BLK = 8
    NB = N // BLK
    NBP = 1280
    m = _seg_scan(h.astype(bf16), batch_col, N, 3)

    m3 = jnp.reshape(m, (NB, BLK, H))
    id3 = jnp.reshape(batch_col, (NB, BLK, 1))
    sub_iota = jax.lax.broadcasted_iota(jnp.int32, (NB, BLK, 1), 1)
    is_last = sub_iota == (BLK - 1)
    SENT = bf16(-3e38)
    tails = jnp.max(jnp.where(is_last, m3, SENT), axis=1)
    tail_ids = jnp.max(jnp.where(is_last, id3, -1), axis=1)
    tails = jnp.concatenate(
        [tails, jnp.full((NBP - NB, H), SENT, bf16)], axis=0)
    tail_ids = jnp.concatenate(
        [tail_ids, jnp.full((NBP - NB, 1), -1, jnp.int32)], axis=0)
    tails = _seg_scan(tails, tail_ids, NBP, 11)

    col_iota = jax.lax.broadcasted_iota(jnp.int32, (1, N), 1)
    sel = ((col_iota == ends) & (counts > 0.0))
    g_end = jnp.dot(sel.astype(bf16), m,
                    preferred_element_type=f32)

    tmask = (col_iota % BLK == (BLK - 1)).astype(f32)
    counts_t = jnp.sum(eq * tmask, axis=1, keepdims=True)
    ends_t = jnp.dot(tri, counts_t,
                     preferred_element_type=f32).astype(jnp.int32) - 1
    colb_iota = jax.lax.broadcasted_iota(jnp.int32, (1, NBP), 1)
    sel_t = ((colb_iota == ends_t) & (counts_t > 0.0))
    g_tail = jnp.dot(sel_t.astype(bf16), tails,
                     preferred_element_type=f32)
    g_tail = jnp.where(counts_t > 0.0, g_tail, NEG_INF)

    x_max = jnp.where(counts > 0.0, jnp.maximum(g_end, g_tail), NEG_INF)

    x_mean = x_sum / jnp.maximum(counts, 1.0)

    x_global = jnp.concatenate([x_mean, x_max, x_sum], axis=1)

    z = _dot_rt(x_global, wc1_ref[...]) + jax.lax.slice(pk, (137, 0), (138, H))
    z = jnp.maximum(z, 0.0)
    z = (_dot_rt(z, jax.lax.slice(pk, (0, 0), (128, H))) +
         jax.lax.slice(pk, (138, 0), (139, H // 2)))
    z = jnp.maximum(z, 0.0)
    z = (_dot_rt(z, jax.lax.slice(pk, (128, 0), (130, H // 2))) +
         jax.lax.slice(pk, (139, 0), (140, OUT)))
    out_ref[...] = z


@jax.jit
def _run(x, batch, W1, b1, Wc1, bc1, Wc2, bc2, Wc3, bc3):
    batch_col = batch.reshape(N, 1)
    batch_row = batch.reshape(1, N)
    pack = jnp.concatenate([
        Wc2,
        jnp.pad(Wc3, ((0, 6), (0, H - H // 2))),
        b1.reshape(1, H),
        bc1.reshape(1, H),
        jnp.pad(bc2.reshape(1, H // 2), ((0, 0), (0, H - H // 2))),
        jnp.pad(bc3.reshape(1, OUT), ((0, 0), (0, H - OUT))),
        jnp.zeros((4, H), jnp.float32),
    ], axis=0)
    return pl.pallas_call(
        _fused_kernel,
        out_shape=jax.ShapeDtypeStruct((G, OUT), jnp.float32),
    )(x, batch_col, batch_row, W1, Wc1, pack)


def kernel(x, edge_index, batch, W1, b1, Wc1, bc1, Wc2, bc2, Wc3, bc3):
    del edge_index
    return _run(x, batch, W1, b1, Wc1, bc1, Wc2, bc2, Wc3, bc3)


# speedup vs baseline: 1.1577x; 1.1577x over previous
import math

import jax
import jax.numpy as jnp
from jax.experimental import pallas as pl

N = 10000
D = 256
H = 256
G = 64
OUT = 2
R = 8
B = N // R
NEG_INF = float("-inf")


def _dot_rt(a, b):
    return jax.lax.dot_general(a, b, (((1,), (1,)), ((), ())),
                               preferred_element_type=jnp.float32)


def _seg_scan(vals, ids, nrows, nsteps):
    f32 = jnp.float32
    w = vals.shape[1]
    for k in range(nsteps):
        s = 1 << k
        same = (jax.lax.slice(ids, (0, 0), (nrows - s, 1)) ==
                jax.lax.slice(ids, (s, 0), (nrows, 1)))
        prev = jax.lax.slice(vals, (0, 0), (nrows - s, w))
        cur = jax.lax.slice(vals, (s, 0), (nrows, w))
        rest = jnp.maximum(cur, jnp.where(same, prev, NEG_INF))
        vals = jnp.concatenate(
            [jax.lax.slice(vals, (0, 0), (s, w)), rest], axis=0)
    return vals


def _fused_kernel(x_ref, batch_col_ref, batch_row_ref,
                  w1_ref, b1_ref, wc1_ref, bc1_ref,
                  wc2_ref, bc2_ref, wc3_ref, bc3_ref,
                  out_ref):
    f32 = jnp.float32

    h = _dot_rt(x_ref[...], w1_ref[...]) + jnp.reshape(b1_ref[...], (1, H))

    batch_col = batch_col_ref[...]
    batch_row = batch_row_ref[...]

    seg_iota = jax.lax.broadcasted_iota(jnp.int32, (G, 1), 0)
    eq = (batch_row == seg_iota).astype(f32)
    le = (batch_row <= seg_iota).astype(f32)

    counts = jnp.sum(eq, axis=1, keepdims=True)
    ends = jnp.sum(le, axis=1, keepdims=True).astype(jnp.int32) - 1

    x_sum = jnp.dot(eq, h, preferred_element_type=f32)

    bf16 = jnp.bfloat16
    BLK = 8
    NB = N // BLK
    NBP = 1280
    m = _seg_scan(h.astype(bf16), batch_col, N, 3)

    m3 = jnp.reshape(m, (NB, BLK, H))
    id3 = jnp.reshape(batch_col, (NB, BLK, 1))
    sub_iota = jax.lax.broadcasted_iota(jnp.int32, (NB, BLK, 1), 1)
    is_last = sub_iota == (BLK - 1)
    SENT = bf16(-3e38)
    tails = jnp.max(jnp.where(is_last, m3, SENT), axis=1)
    tail_ids = jnp.max(jnp.where(is_last, id3, -1), axis=1)
    tails = jnp.concatenate(
        [tails, jnp.full((NBP - NB, H), SENT, bf16)], axis=0)
    tail_ids = jnp.concatenate(
        [tail_ids, jnp.full((NBP - NB, 1), -1, jnp.int32)], axis=0)
    tails = _seg_scan(tails, tail_ids, NBP, 11)

    col_iota = jax.lax.broadcasted_iota(jnp.int32, (1, N), 1)
    sel = ((col_iota == ends) & (counts > 0.0))
    g_end = jnp.dot(sel.astype(bf16), m,
                    preferred_element_type=f32)

    tmask = (col_iota % BLK == (BLK - 1)).astype(f32)
    counts_t = jnp.sum(eq * tmask, axis=1, keepdims=True)
    ends_t = jnp.sum(le * tmask, axis=1, keepdims=True).astype(jnp.int32) - 1
    colb_iota = jax.lax.broadcasted_iota(jnp.int32, (1, NBP), 1)
    sel_t = ((colb_iota == ends_t) & (counts_t > 0.0))
    g_tail = jnp.dot(sel_t.astype(bf16), tails,
                     preferred_element_type=f32)
    g_tail = jnp.where(counts_t > 0.0, g_tail, NEG_INF)

    x_max = jnp.where(counts > 0.0, jnp.maximum(g_end, g_tail), NEG_INF)

    x_mean = x_sum / jnp.maximum(counts, 1.0)

    x_global = jnp.concatenate([x_mean, x_max, x_sum], axis=1)

    z = _dot_rt(x_global, wc1_ref[...]) + jnp.reshape(bc1_ref[...], (1, H))
    z = jnp.maximum(z, 0.0)
    z = _dot_rt(z, wc2_ref[...]) + jnp.reshape(bc2_ref[...], (1, H // 2))
    z = jnp.maximum(z, 0.0)
    z = _dot_rt(z, wc3_ref[...]) + jnp.reshape(bc3_ref[...], (1, OUT))
    out_ref[...] = z


@jax.jit
def _run(x, batch, W1, b1, Wc1, bc1, Wc2, bc2, Wc3, bc3):
    batch_col = batch.reshape(N, 1)
    batch_row = batch.reshape(1, N)
    return pl.pallas_call(
        _fused_kernel,
        out_shape=jax.ShapeDtypeStruct((G, OUT), jnp.float32),
    )(x, batch_col, batch_row,
      W1, b1, Wc1, bc1, Wc2, bc2, Wc3, bc3)


def kernel(x, edge_index, batch, W1, b1, Wc1, bc1, Wc2, bc2, Wc3, bc3):
    del edge_index
    return _run(x, batch, W1, b1, Wc1, bc1, Wc2, bc2, Wc3, bc3)
